# baseline (speedup 1.0000x reference)
import jax
import jax.numpy as jnp
from jax.experimental import pallas as pl
from jax.experimental.pallas import tpu as pltpu


def _support_kernel(x_ref, w_ref, out_ref):
    out_ref[...] = jnp.dot(
        x_ref[...], w_ref[...], preferred_element_type=jnp.float32
    )


def _spmm_kernel(adj_ref, s_ref, b_ref, out_ref):
    acc = jnp.dot(
        adj_ref[...], s_ref[...], preferred_element_type=jnp.float32
    )
    out_ref[...] = jnp.maximum(acc + b_ref[...], 0.0)


def kernel(x, adj, W, b):
    n, nfeat = x.shape
    nhid = W.shape[1]

    support = pl.pallas_call(
        _support_kernel,
        out_shape=jax.ShapeDtypeStruct((n, nhid), jnp.float32),
    )(x, W)

    bm = 400
    out = pl.pallas_call(
        _spmm_kernel,
        grid=(n // bm,),
        in_specs=[
            pl.BlockSpec((bm, n), lambda i: (i, 0)),
            pl.BlockSpec((n, nhid), lambda i: (0, 0)),
            pl.BlockSpec((1, nhid), lambda i: (0, 0)),
        ],
        out_specs=pl.BlockSpec((bm, nhid), lambda i: (i, 0)),
        out_shape=jax.ShapeDtypeStruct((n, nhid), jnp.float32),
        compiler_params=pltpu.CompilerParams(
            dimension_semantics=("parallel",),
        ),
    )(adj, support, b.reshape(1, nhid))
    return out


# speedup vs baseline: 1.0688x; 1.0688x over previous
import jax
import jax.numpy as jnp
from jax.experimental import pallas as pl
from jax.experimental.pallas import tpu as pltpu


def _gcn_kernel(adj0_ref, adj1_ref, x_ref, w_ref, b_ref, out_ref, s_ref):
    @pl.when(pl.program_id(0) == 0)
    def _():
        s_ref[...] = jnp.dot(
            x_ref[...], w_ref[...], preferred_element_type=jnp.float32
        )

    bm = adj0_ref.shape[0]
    acc0 = jnp.dot(
        adj0_ref[...], s_ref[...], preferred_element_type=jnp.float32
    )
    out_ref[0:bm, :] = jnp.maximum(acc0 + b_ref[...], 0.0)
    acc1 = jnp.dot(
        adj1_ref[...], s_ref[...], preferred_element_type=jnp.float32
    )
    out_ref[bm : 2 * bm, :] = jnp.maximum(acc1 + b_ref[...], 0.0)


def kernel(x, adj, W, b):
    n, nfeat = x.shape
    nhid = W.shape[1]

    bm = 200
    out = pl.pallas_call(
        _gcn_kernel,
        grid=(n // (2 * bm),),
        in_specs=[
            pl.BlockSpec((bm, n), lambda i: (2 * i, 0)),
            pl.BlockSpec((bm, n), lambda i: (2 * i + 1, 0)),
            pl.BlockSpec((n, nfeat), lambda i: (0, 0)),
            pl.BlockSpec((nfeat, nhid), lambda i: (0, 0)),
            pl.BlockSpec((1, nhid), lambda i: (0, 0)),
        ],
        out_specs=pl.BlockSpec((2 * bm, nhid), lambda i: (i, 0)),
        out_shape=jax.ShapeDtypeStruct((n, nhid), jnp.float32),
        scratch_shapes=[pltpu.VMEM((n, nhid), jnp.float32)],
    )(adj, adj, x, W, b.reshape(1, nhid))
    return out
